# baseline (speedup 1.0000x reference)
import functools

import jax
import jax.numpy as jnp
from jax import lax
from jax.experimental import pallas as pl
from jax.experimental.pallas import tpu as pltpu
from jax.experimental.pallas import tpu_sc as plsc

_NC, _NS, _L = 2, 16, 16
_NW = _NC * _NS
_XB = 2

_B, _N, _M, _D = 8, 2048, 2048, 3
_R = 1408
_NT = 128
_WPB = _NW // _B
_QN = (_N - _R) // _WPB
_YC = _M // _L


def _sc_body(x_hbm, yt_hbm, colmin_hbm, rowsum_hbm,
             x_v, y0_v, y1_v, y2_v, cm_v, rs_v):
    wid = lax.axis_index("s") * _NC + lax.axis_index("c")
    b = wid // _WPB
    q = wid % _WPB

    pltpu.sync_copy(x_hbm.at[pl.ds((b * _N + _R + q * _QN) * _D, _QN * _D)],
                    x_v.at[pl.ds(0, _QN * _D)])
    for d, yr in enumerate((y0_v, y1_v, y2_v)):
        pltpu.sync_copy(yt_hbm.at[pl.ds((b * _D + d) * _M, _M)], yr)

    inf16 = jnp.full((_L,), jnp.inf, jnp.float32)

    def init_cm(c, carry):
        cm_v[pl.ds(c * _L, _L)] = inf16
        return carry
    lax.fori_loop(0, _YC, init_cm, 0)

    def xblk_body(xb, rs_acc):
        xv = x_v[pl.ds(xb * (_XB * _D), _L)]
        xbc = [[jnp.full((_L,), xv[j * _D + d]) for d in range(_D)]
               for j in range(_XB)]

        @plsc.parallel_loop(0, _M, _L, unroll=4, carry=(inf16,) * _XB)
        def rms(off, rms_in):
            y0 = y0_v[pl.ds(off, _L)]
            y1 = y1_v[pl.ds(off, _L)]
            y2 = y2_v[pl.ds(off, _L)]
            dists = []
            for j in range(_XB):
                dj = (jnp.abs(xbc[j][0] - y0) + jnp.abs(xbc[j][1] - y1)
                      + jnp.abs(xbc[j][2] - y2))
                dists.append(dj)
            out = tuple(jnp.minimum(rms_in[j], dists[j]) for j in range(_XB))
            cmin = functools.reduce(jnp.minimum, dists)
            cm_v[pl.ds(off, _L)] = jnp.minimum(cm_v[pl.ds(off, _L)], cmin)
            return out

        for j in range(_XB):
            rs_acc = rs_acc + plsc.cummax(-rms[j])
        return rs_acc

    rs_acc = lax.fori_loop(0, _QN // _XB, xblk_body,
                           jnp.zeros((_L,), jnp.float32))
    rs_v[...] = rs_acc
    pltpu.sync_copy(rs_v, rowsum_hbm.at[pl.ds(wid * _L, _L)])
    pltpu.sync_copy(cm_v, colmin_hbm.at[pl.ds(wid * _M, _M)])


_sc_chamfer = functools.partial(
    pl.kernel,
    out_type=(jax.ShapeDtypeStruct((_NW * _M,), jnp.float32),
              jax.ShapeDtypeStruct((_NW * _L,), jnp.float32)),
    mesh=plsc.VectorSubcoreMesh(core_axis_name="c", subcore_axis_name="s"),
    compiler_params=pltpu.CompilerParams(needs_layout_passes=False),
    scratch_types=(
        pltpu.VMEM((_QN * _D + _L,), jnp.float32),
        pltpu.VMEM((_M,), jnp.float32),
        pltpu.VMEM((_M,), jnp.float32),
        pltpu.VMEM((_M,), jnp.float32),
        pltpu.VMEM((_M,), jnp.float32),
        pltpu.VMEM((_L,), jnp.float32),
    ),
)(_sc_body)


def _tc_body(x_ref, yt_ref, cm_out_ref, sx_ref, colmin_ref, accs_ref, *, nb):
    i = pl.program_id(1)
    b = pl.program_id(0)

    xb = x_ref[0]
    yt = yt_ref[0]

    acc = jnp.abs(xb[:, 0:1] - yt[0:1, :])
    acc = acc + jnp.abs(xb[:, 1:2] - yt[1:2, :])
    acc = acc + jnp.abs(xb[:, 2:3] - yt[2:3, :])

    @pl.when(jnp.logical_and(b == 0, i == 0))
    def _init():
        accs_ref[0, 0] = 0.0

    @pl.when(i == 0)
    def _reset_colmin():
        colmin_ref[...] = jnp.full_like(colmin_ref, jnp.inf)

    row_min = jnp.min(acc, axis=1)
    accs_ref[0, 0] += jnp.sum(row_min)

    colmin_ref[...] = jnp.minimum(colmin_ref[...],
                                  jnp.min(acc, axis=0, keepdims=True))

    @pl.when(i == nb - 1)
    def _emit_colmin():
        cm_out_ref[0] = colmin_ref[...]

    @pl.when(jnp.logical_and(b == _B - 1, i == nb - 1))
    def _emit_sx():
        sx_ref[0, 0] = accs_ref[0, 0]


def _tc_chamfer(x, yt):
    nb = _R // _NT
    return pl.pallas_call(
        functools.partial(_tc_body, nb=nb),
        grid=(_B, nb),
        in_specs=[
            pl.BlockSpec((1, _NT, _D), lambda b, i: (b, i, 0)),
            pl.BlockSpec((1, _D, _M), lambda b, i: (b, 0, 0)),
        ],
        out_specs=[
            pl.BlockSpec((1, 1, _M), lambda b, i: (b, 0, 0)),
            pl.BlockSpec((1, 1), lambda b, i: (0, 0), memory_space=pltpu.SMEM),
        ],
        out_shape=[
            jax.ShapeDtypeStruct((_B, 1, _M), jnp.float32),
            jax.ShapeDtypeStruct((1, 1), jnp.float32),
        ],
        scratch_shapes=[
            pltpu.VMEM((1, _M), jnp.float32),
            pltpu.SMEM((1, 1), jnp.float32),
        ],
    )(x, yt)


def _combine_body(cm_sc_ref, rs_ref, cm_tc_ref, sx_tc_ref, out_ref):
    ymin_sc = jnp.min(cm_sc_ref[...], axis=1)
    ymin = jnp.minimum(ymin_sc, jnp.min(cm_tc_ref[...], axis=1))
    sy = jnp.sum(ymin)
    sx = sx_tc_ref[0, 0] - jnp.sum(rs_ref[:, _L - 1:_L])
    out_ref[0, 0] = sx / (_B * _N) + sy / (_B * _M)


def _combine(colmin_sc, rowsum_sc, colmin_tc, sx_tc):
    out = pl.pallas_call(
        _combine_body,
        in_specs=[
            pl.BlockSpec((_B, _WPB, _M), lambda: (0, 0, 0)),
            pl.BlockSpec((_NW, _L), lambda: (0, 0)),
            pl.BlockSpec((_B, 1, _M), lambda: (0, 0, 0)),
            pl.BlockSpec((1, 1), lambda: (0, 0), memory_space=pltpu.SMEM),
        ],
        out_specs=pl.BlockSpec(memory_space=pltpu.SMEM),
        out_shape=jax.ShapeDtypeStruct((1, 1), jnp.float32),
    )(colmin_sc.reshape(_B, _WPB, _M), rowsum_sc.reshape(_NW, _L),
      colmin_tc, sx_tc)
    return out[0, 0]


def kernel(x, y):
    yt = jnp.transpose(y, (0, 2, 1))
    colmin_sc, rowsum_sc = _sc_chamfer(x.reshape(-1), yt.reshape(-1))
    colmin_tc, sx_tc = _tc_chamfer(x, yt)
    return _combine(colmin_sc, rowsum_sc, colmin_tc, sx_tc)


# speedup vs baseline: 1.3523x; 1.3523x over previous
import functools

import jax
import jax.numpy as jnp
from jax import lax
from jax.experimental import pallas as pl
from jax.experimental.pallas import tpu as pltpu
from jax.experimental.pallas import tpu_sc as plsc

_NC, _NS, _L = 2, 16, 16
_NW = _NC * _NS
_XB = 2

_B, _N, _M, _D = 8, 2048, 2048, 3
_R = 1536
_NT = 256
_WPB = _NW // _B
_QN = (_N - _R) // _WPB
_YC = _M // _L


def _sc_body(xt_hbm, yt_hbm, colmin_hbm, rowsum_hbm,
             x0_v, x1_v, x2_v, y0_v, y1_v, y2_v, cm_v, rs_v):
    wid = lax.axis_index("s") * _NC + lax.axis_index("c")
    b = wid // _WPB
    q = wid % _WPB

    for d, xr in enumerate((x0_v, x1_v, x2_v)):
        pltpu.sync_copy(
            xt_hbm.at[pl.ds((b * _D + d) * _N + _R + q * _QN, _QN)],
            xr.at[pl.ds(0, _QN)])
        pltpu.sync_copy(yt_hbm.at[pl.ds((b * _D + d) * _M, _M)],
                        (y0_v, y1_v, y2_v)[d])

    inf16 = jnp.full((_L,), jnp.inf, jnp.float32)

    def init_cm(c, carry):
        cm_v[pl.ds(c * _L, _L)] = inf16
        return carry
    lax.fori_loop(0, _YC, init_cm, 0)

    def xblk_body(xb, rs_acc):
        base = xb * _XB
        xv = [xr[pl.ds(base, _L)] for xr in (x0_v, x1_v, x2_v)]
        xbc = [[jnp.full((_L,), xv[d][j]) for d in range(_D)]
               for j in range(_XB)]

        @plsc.parallel_loop(0, _M, _L, unroll=4, carry=(inf16,) * _XB)
        def rms(off, rms_in):
            y0 = y0_v[pl.ds(off, _L)]
            y1 = y1_v[pl.ds(off, _L)]
            y2 = y2_v[pl.ds(off, _L)]
            dists = []
            for j in range(_XB):
                dj = (jnp.abs(xbc[j][0] - y0) + jnp.abs(xbc[j][1] - y1)
                      + jnp.abs(xbc[j][2] - y2))
                dists.append(dj)
            out = tuple(jnp.minimum(rms_in[j], dists[j]) for j in range(_XB))
            cmin = functools.reduce(jnp.minimum, dists)
            cm_v[pl.ds(off, _L)] = jnp.minimum(cm_v[pl.ds(off, _L)], cmin)
            return out

        for j in range(_XB):
            rs_acc = rs_acc + plsc.cummax(-rms[j])
        return rs_acc

    rs_acc = lax.fori_loop(0, _QN // _XB, xblk_body,
                           jnp.zeros((_L,), jnp.float32))
    rs_v[...] = rs_acc
    pltpu.sync_copy(rs_v, rowsum_hbm.at[pl.ds(wid * _L, _L)])
    pltpu.sync_copy(cm_v, colmin_hbm.at[pl.ds(wid * _M, _M)])


_sc_chamfer = functools.partial(
    pl.kernel,
    out_type=(jax.ShapeDtypeStruct((_NW * _M,), jnp.float32),
              jax.ShapeDtypeStruct((_NW * _L,), jnp.float32)),
    mesh=plsc.VectorSubcoreMesh(core_axis_name="c", subcore_axis_name="s"),
    compiler_params=pltpu.CompilerParams(needs_layout_passes=False),
    scratch_types=(
        pltpu.VMEM((_QN + _L,), jnp.float32),
        pltpu.VMEM((_QN + _L,), jnp.float32),
        pltpu.VMEM((_QN + _L,), jnp.float32),
        pltpu.VMEM((_M,), jnp.float32),
        pltpu.VMEM((_M,), jnp.float32),
        pltpu.VMEM((_M,), jnp.float32),
        pltpu.VMEM((_M,), jnp.float32),
        pltpu.VMEM((_L,), jnp.float32),
    ),
)(_sc_body)


def _tc_body(x_ref, yt_ref, cm_out_ref, sx_ref, colmin_ref, accs_ref, *, nb):
    i = pl.program_id(1)
    b = pl.program_id(0)

    xb = x_ref[0]
    yt = yt_ref[0]

    acc = jnp.abs(xb[:, 0:1] - yt[0:1, :])
    acc = acc + jnp.abs(xb[:, 1:2] - yt[1:2, :])
    acc = acc + jnp.abs(xb[:, 2:3] - yt[2:3, :])

    @pl.when(jnp.logical_and(b == 0, i == 0))
    def _init():
        accs_ref[0, 0] = 0.0

    @pl.when(i == 0)
    def _reset_colmin():
        colmin_ref[...] = jnp.full_like(colmin_ref, jnp.inf)

    row_min = jnp.min(acc, axis=1)
    accs_ref[0, 0] += jnp.sum(row_min)

    colmin_ref[...] = jnp.minimum(colmin_ref[...],
                                  jnp.min(acc, axis=0, keepdims=True))

    @pl.when(i == nb - 1)
    def _emit_colmin():
        cm_out_ref[0] = colmin_ref[...]

    @pl.when(jnp.logical_and(b == _B - 1, i == nb - 1))
    def _emit_sx():
        sx_ref[0, 0] = accs_ref[0, 0]


def _tc_chamfer(x, yt):
    nb = _R // _NT
    return pl.pallas_call(
        functools.partial(_tc_body, nb=nb),
        grid=(_B, nb),
        in_specs=[
            pl.BlockSpec((1, _NT, _D), lambda b, i: (b, i, 0)),
            pl.BlockSpec((1, _D, _M), lambda b, i: (b, 0, 0)),
        ],
        out_specs=[
            pl.BlockSpec((1, 1, _M), lambda b, i: (b, 0, 0)),
            pl.BlockSpec((1, 1), lambda b, i: (0, 0), memory_space=pltpu.SMEM),
        ],
        out_shape=[
            jax.ShapeDtypeStruct((_B, 1, _M), jnp.float32),
            jax.ShapeDtypeStruct((1, 1), jnp.float32),
        ],
        scratch_shapes=[
            pltpu.VMEM((1, _M), jnp.float32),
            pltpu.SMEM((1, 1), jnp.float32),
        ],
    )(x, yt)


def _combine_body(cm_sc_ref, rs_ref, cm_tc_ref, sx_tc_ref, out_ref):
    cm_tc = cm_tc_ref[...]
    sy = jnp.float32(0.0)
    for b in range(_B):
        mins = [cm_sc_ref[pl.ds((b * _WPB + q) * _M, _M)]
                for q in range(_WPB)]
        mb = functools.reduce(jnp.minimum, mins)
        mb = jnp.minimum(mb, cm_tc[b, 0])
        sy = sy + jnp.sum(mb)
    rs = rs_ref[...]
    lane = lax.broadcasted_iota(jnp.int32, (_NW * _L,), 0)
    sx = sx_tc_ref[0, 0] - jnp.sum(jnp.where(lane % _L == _L - 1, rs, 0.0))
    out_ref[0, 0] = sx / (_B * _N) + sy / (_B * _M)


def _combine(colmin_sc, rowsum_sc, colmin_tc, sx_tc):
    out = pl.pallas_call(
        _combine_body,
        in_specs=[
            pl.BlockSpec((_NW * _M,), lambda: (0,)),
            pl.BlockSpec((_NW * _L,), lambda: (0,)),
            pl.BlockSpec((_B, 1, _M), lambda: (0, 0, 0)),
            pl.BlockSpec((1, 1), lambda: (0, 0), memory_space=pltpu.SMEM),
        ],
        out_specs=pl.BlockSpec(memory_space=pltpu.SMEM),
        out_shape=jax.ShapeDtypeStruct((1, 1), jnp.float32),
    )(colmin_sc, rowsum_sc, colmin_tc, sx_tc)
    return out[0, 0]


def kernel(x, y):
    xt = jnp.transpose(x, (0, 2, 1)).reshape(-1)
    yt = jnp.transpose(y, (0, 2, 1))
    colmin_sc, rowsum_sc = _sc_chamfer(xt, yt.reshape(-1))
    colmin_tc, sx_tc = _tc_chamfer(x, yt)
    return _combine(colmin_sc, rowsum_sc, colmin_tc, sx_tc)


# speedup vs baseline: 1.5812x; 1.1693x over previous
import functools

import jax
import jax.numpy as jnp
from jax import lax
from jax.experimental import pallas as pl
from jax.experimental.pallas import tpu as pltpu
from jax.experimental.pallas import tpu_sc as plsc

_NC, _NS, _L = 2, 16, 16
_NW = _NC * _NS
_XB = 2

_B, _N, _M, _D = 8, 2048, 2048, 3
_R = 1536
_NT = 256
_WPB = _NW // _B
_QN = (_N - _R) // _WPB
_YC = _M // _L


def _sc_body(xt_hbm, yt_hbm, colmin_hbm, rowsum_hbm,
             x0_v, x1_v, x2_v, y0_v, y1_v, y2_v, cm_v, rs_v):
    wid = lax.axis_index("s") * _NC + lax.axis_index("c")
    b = wid // _WPB
    q = wid % _WPB

    for d, (xr, yr) in enumerate(((x0_v, y0_v), (x1_v, y1_v), (x2_v, y2_v))):
        pltpu.sync_copy(
            xt_hbm.at[pl.ds((b * _D + d) * _N + _R + q * _QN, _QN)],
            xr.at[pl.ds(0, _QN)])
        pltpu.sync_copy(yt_hbm.at[pl.ds((b * _D + d) * _M, _M)], yr)

    inf16 = jnp.full((_L,), jnp.inf, jnp.float32)

    def init_cm(c, carry):
        cm_v[pl.ds(c * _L, _L)] = inf16
        return carry
    lax.fori_loop(0, _YC, init_cm, 0)

    def xblk_body(xb, rs_acc):
        base = xb * _XB
        xv = [xr[pl.ds(base, _L)] for xr in (x0_v, x1_v, x2_v)]
        xbc = [[jnp.full((_L,), xv[d][j]) for d in range(_D)]
               for j in range(_XB)]

        @plsc.parallel_loop(0, _M, _L, unroll=4, carry=(inf16,) * _XB)
        def rms(off, rms_in):
            y0 = y0_v[pl.ds(off, _L)]
            y1 = y1_v[pl.ds(off, _L)]
            y2 = y2_v[pl.ds(off, _L)]
            dists = []
            for j in range(_XB):
                dj = (jnp.abs(xbc[j][0] - y0) + jnp.abs(xbc[j][1] - y1)
                      + jnp.abs(xbc[j][2] - y2))
                dists.append(dj)
            out = tuple(jnp.minimum(rms_in[j], dists[j]) for j in range(_XB))
            cmin = functools.reduce(jnp.minimum, dists)
            cm_v[pl.ds(off, _L)] = jnp.minimum(cm_v[pl.ds(off, _L)], cmin)
            return out

        for j in range(_XB):
            rs_acc = rs_acc + plsc.cummax(-rms[j])
        return rs_acc

    rs_acc = lax.fori_loop(0, _QN // _XB, xblk_body,
                           jnp.zeros((_L,), jnp.float32))
    rs_v[...] = rs_acc
    pltpu.sync_copy(rs_v, rowsum_hbm.at[pl.ds(wid * _L, _L)])
    pltpu.sync_copy(cm_v, colmin_hbm.at[pl.ds(wid * _M, _M)])


_sc_chamfer = functools.partial(
    pl.kernel,
    out_type=(jax.ShapeDtypeStruct((_NW * _M,), jnp.float32),
              jax.ShapeDtypeStruct((_NW * _L,), jnp.float32)),
    mesh=plsc.VectorSubcoreMesh(core_axis_name="c", subcore_axis_name="s"),
    compiler_params=pltpu.CompilerParams(needs_layout_passes=False),
    scratch_types=(
        pltpu.VMEM((_QN + _L,), jnp.float32),
        pltpu.VMEM((_QN + _L,), jnp.float32),
        pltpu.VMEM((_QN + _L,), jnp.float32),
        pltpu.VMEM((_M,), jnp.float32),
        pltpu.VMEM((_M,), jnp.float32),
        pltpu.VMEM((_M,), jnp.float32),
        pltpu.VMEM((_M,), jnp.float32),
        pltpu.VMEM((_L,), jnp.float32),
    ),
)(_sc_body)


def _tc_body(x_ref, yt_ref, cm_out_ref, sx_ref, colmin_ref, accs_ref, *, nb):
    i = pl.program_id(1)
    b = pl.program_id(0)

    xb = x_ref[0].astype(jnp.bfloat16)
    yt = yt_ref[0].astype(jnp.bfloat16)

    acc = jnp.abs(xb[:, 0:1] - yt[0:1, :])
    acc = acc + jnp.abs(xb[:, 1:2] - yt[1:2, :])
    acc = acc + jnp.abs(xb[:, 2:3] - yt[2:3, :])

    @pl.when(jnp.logical_and(b == 0, i == 0))
    def _init():
        accs_ref[0, 0] = 0.0

    @pl.when(i == 0)
    def _reset_colmin():
        colmin_ref[...] = jnp.full_like(colmin_ref, jnp.inf)

    row_min = jnp.min(acc, axis=1).astype(jnp.float32)
    accs_ref[0, 0] += jnp.sum(row_min)

    colmin_ref[...] = jnp.minimum(
        colmin_ref[...],
        jnp.min(acc, axis=0, keepdims=True).astype(jnp.float32))

    @pl.when(i == nb - 1)
    def _emit_colmin():
        cm_out_ref[0] = colmin_ref[...]

    @pl.when(jnp.logical_and(b == _B - 1, i == nb - 1))
    def _emit_sx():
        sx_ref[0, 0] = accs_ref[0, 0]


def _tc_chamfer(x, yt):
    nb = _R // _NT
    return pl.pallas_call(
        functools.partial(_tc_body, nb=nb),
        grid=(_B, nb),
        in_specs=[
            pl.BlockSpec((1, _NT, _D), lambda b, i: (b, i, 0)),
            pl.BlockSpec((1, _D, _M), lambda b, i: (b, 0, 0)),
        ],
        out_specs=[
            pl.BlockSpec((1, 1, _M), lambda b, i: (b, 0, 0)),
            pl.BlockSpec((1, 1), lambda b, i: (0, 0), memory_space=pltpu.SMEM),
        ],
        out_shape=[
            jax.ShapeDtypeStruct((_B, 1, _M), jnp.float32),
            jax.ShapeDtypeStruct((1, 1), jnp.float32),
        ],
        scratch_shapes=[
            pltpu.VMEM((1, _M), jnp.float32),
            pltpu.SMEM((1, 1), jnp.float32),
        ],
    )(x, yt)


def _combine_body(cm_sc_ref, rs_ref, cm_tc_ref, sx_tc_ref, out_ref):
    cm_tc = cm_tc_ref[...]
    sy = jnp.float32(0.0)
    for b in range(_B):
        mins = [cm_sc_ref[pl.ds((b * _WPB + q) * _M, _M)]
                for q in range(_WPB)]
        mb = functools.reduce(jnp.minimum, mins)
        mb = jnp.minimum(mb, cm_tc[b, 0])
        sy = sy + jnp.sum(mb)
    rs = rs_ref[...]
    lane = lax.broadcasted_iota(jnp.int32, (_NW * _L,), 0)
    sx = sx_tc_ref[0, 0] - jnp.sum(jnp.where(lane % _L == _L - 1, rs, 0.0))
    out_ref[0, 0] = sx / (_B * _N) + sy / (_B * _M)


def _combine(colmin_sc, rowsum_sc, colmin_tc, sx_tc):
    out = pl.pallas_call(
        _combine_body,
        in_specs=[
            pl.BlockSpec((_NW * _M,), lambda: (0,)),
            pl.BlockSpec((_NW * _L,), lambda: (0,)),
            pl.BlockSpec((_B, 1, _M), lambda: (0, 0, 0)),
            pl.BlockSpec((1, 1), lambda: (0, 0), memory_space=pltpu.SMEM),
        ],
        out_specs=pl.BlockSpec(memory_space=pltpu.SMEM),
        out_shape=jax.ShapeDtypeStruct((1, 1), jnp.float32),
    )(colmin_sc, rowsum_sc, colmin_tc, sx_tc)
    return out[0, 0]


def kernel(x, y):
    xt_flat = jnp.transpose(x, (0, 2, 1)).reshape(-1)
    yt_flat = jnp.transpose(y, (0, 2, 1)).reshape(-1)
    yt = jnp.transpose(y, (0, 2, 1))
    colmin_sc, rowsum_sc = _sc_chamfer(xt_flat, yt_flat)
    colmin_tc, sx_tc = _tc_chamfer(x, yt)
    return _combine(colmin_sc, rowsum_sc, colmin_tc, sx_tc)


# speedup vs baseline: 1.6215x; 1.0255x over previous
import functools

import jax
import jax.numpy as jnp
from jax import lax
from jax.experimental import pallas as pl
from jax.experimental.pallas import tpu as pltpu
from jax.experimental.pallas import tpu_sc as plsc

_NC, _NS, _L = 2, 16, 16
_NW = _NC * _NS
_XB = 2

_B, _N, _M, _D = 8, 2048, 2048, 3
_R = 1536
_NT = 256
_WPB = _NW // _B
_QN = (_N - _R) // _WPB
_YC = _M // _L


_L2 = 2 * _L


def _sc_body(xt_hbm, yt_hbm, colmin_hbm, rowsum_hbm,
             x0_v, x1_v, x2_v, y0_v, y1_v, y2_v, cm_v, rs_v):
    wid = lax.axis_index("s") * _NC + lax.axis_index("c")
    b = wid // _WPB
    q = wid % _WPB

    for d, (xr, yr) in enumerate(((x0_v, y0_v), (x1_v, y1_v), (x2_v, y2_v))):
        pltpu.sync_copy(
            xt_hbm.at[pl.ds((b * _D + d) * _N + _R + q * _QN, _QN)],
            xr.at[pl.ds(0, _QN)])
        pltpu.sync_copy(yt_hbm.at[pl.ds((b * _D + d) * _M, _M)], yr)

    inf32 = jnp.full((_L2,), jnp.inf, jnp.bfloat16)

    def init_cm(c, carry):
        cm_v[pl.ds(c * _L2, _L2)] = inf32
        return carry
    lax.fori_loop(0, _M // _L2, init_cm, 0)

    def xblk_body(xb, rs_acc):
        base = xb * _XB
        xv = [xr[pl.ds(base, _L)] for xr in (x0_v, x1_v, x2_v)]
        xbc = [[plsc.pack(jnp.full((_L,), xv[d][j]),
                          jnp.full((_L,), xv[d][j]),
                          format=plsc.PackFormat.INTERLEAVED)
                for d in range(_D)]
               for j in range(_XB)]

        @plsc.parallel_loop(0, _M, _L2, unroll=4,
                            carry=(inf32,) * _XB)
        def rms(off, rms_in):
            y0 = y0_v[pl.ds(off, _L2)]
            y1 = y1_v[pl.ds(off, _L2)]
            y2 = y2_v[pl.ds(off, _L2)]
            dists = []
            for j in range(_XB):
                dj = (jnp.abs(xbc[j][0] - y0) + jnp.abs(xbc[j][1] - y1)
                      + jnp.abs(xbc[j][2] - y2))
                dists.append(dj)
            out = tuple(jnp.minimum(rms_in[j], dists[j]) for j in range(_XB))
            cmin = functools.reduce(jnp.minimum, dists)
            cm_v[pl.ds(off, _L2)] = jnp.minimum(cm_v[pl.ds(off, _L2)], cmin)
            return out

        for j in range(_XB):
            ha, hb = plsc.unpack(rms[j], format=plsc.PackFormat.INTERLEAVED)
            rs_acc = rs_acc + plsc.cummax(-jnp.minimum(ha, hb))
        return rs_acc

    rs_acc = lax.fori_loop(0, _QN // _XB, xblk_body,
                           jnp.zeros((_L,), jnp.float32))
    rs_v[...] = rs_acc
    pltpu.sync_copy(rs_v, rowsum_hbm.at[pl.ds(wid * _L, _L)])
    pltpu.sync_copy(cm_v, colmin_hbm.at[pl.ds(wid * _M, _M)])


_sc_chamfer = functools.partial(
    pl.kernel,
    out_type=(jax.ShapeDtypeStruct((_NW * _M,), jnp.bfloat16),
              jax.ShapeDtypeStruct((_NW * _L,), jnp.float32)),
    mesh=plsc.VectorSubcoreMesh(core_axis_name="c", subcore_axis_name="s"),
    compiler_params=pltpu.CompilerParams(needs_layout_passes=False),
    scratch_types=(
        pltpu.VMEM((_QN + _L,), jnp.float32),
        pltpu.VMEM((_QN + _L,), jnp.float32),
        pltpu.VMEM((_QN + _L,), jnp.float32),
        pltpu.VMEM((_M,), jnp.bfloat16),
        pltpu.VMEM((_M,), jnp.bfloat16),
        pltpu.VMEM((_M,), jnp.bfloat16),
        pltpu.VMEM((_M,), jnp.bfloat16),
        pltpu.VMEM((_L,), jnp.float32),
    ),
)(_sc_body)


def _tc_body(x_ref, yt_ref, cm_out_ref, sx_ref, colmin_ref, accs_ref, *, nb):
    i = pl.program_id(1)
    b = pl.program_id(0)

    xb = x_ref[0].astype(jnp.bfloat16)
    yt = yt_ref[0].astype(jnp.bfloat16)

    acc = jnp.abs(xb[:, 0:1] - yt[0:1, :])
    acc = acc + jnp.abs(xb[:, 1:2] - yt[1:2, :])
    acc = acc + jnp.abs(xb[:, 2:3] - yt[2:3, :])

    @pl.when(jnp.logical_and(b == 0, i == 0))
    def _init():
        accs_ref[0, 0] = 0.0

    @pl.when(i == 0)
    def _reset_colmin():
        colmin_ref[...] = jnp.full_like(colmin_ref, jnp.inf)

    row_min = jnp.min(acc, axis=1).astype(jnp.float32)
    accs_ref[0, 0] += jnp.sum(row_min)

    colmin_ref[...] = jnp.minimum(
        colmin_ref[...],
        jnp.min(acc, axis=0, keepdims=True).astype(jnp.float32))

    @pl.when(i == nb - 1)
    def _emit_colmin():
        cm_out_ref[0] = colmin_ref[...]

    @pl.when(jnp.logical_and(b == _B - 1, i == nb - 1))
    def _emit_sx():
        sx_ref[0, 0] = accs_ref[0, 0]


def _tc_chamfer(x, yt):
    nb = _R // _NT
    return pl.pallas_call(
        functools.partial(_tc_body, nb=nb),
        grid=(_B, nb),
        in_specs=[
            pl.BlockSpec((1, _NT, _D), lambda b, i: (b, i, 0)),
            pl.BlockSpec((1, _D, _M), lambda b, i: (b, 0, 0)),
        ],
        out_specs=[
            pl.BlockSpec((1, 1, _M), lambda b, i: (b, 0, 0)),
            pl.BlockSpec((1, 1), lambda b, i: (0, 0), memory_space=pltpu.SMEM),
        ],
        out_shape=[
            jax.ShapeDtypeStruct((_B, 1, _M), jnp.float32),
            jax.ShapeDtypeStruct((1, 1), jnp.float32),
        ],
        scratch_shapes=[
            pltpu.VMEM((1, _M), jnp.float32),
            pltpu.SMEM((1, 1), jnp.float32),
        ],
    )(x, yt)


def _combine_body(cm_sc_ref, rs_ref, cm_tc_ref, sx_tc_ref, out_ref):
    cm_tc = cm_tc_ref[...]
    sy = jnp.float32(0.0)
    for b in range(_B):
        mins = [cm_sc_ref[pl.ds((b * _WPB + q) * _M, _M)]
                for q in range(_WPB)]
        mb = functools.reduce(jnp.minimum, mins).astype(jnp.float32)
        mb = jnp.minimum(mb, cm_tc[b, 0])
        sy = sy + jnp.sum(mb)
    rs = rs_ref[...]
    lane = lax.broadcasted_iota(jnp.int32, (_NW * _L,), 0)
    sx = sx_tc_ref[0, 0] - jnp.sum(jnp.where(lane % _L == _L - 1, rs, 0.0))
    out_ref[0, 0] = sx / (_B * _N) + sy / (_B * _M)


def _combine(colmin_sc, rowsum_sc, colmin_tc, sx_tc):
    out = pl.pallas_call(
        _combine_body,
        in_specs=[
            pl.BlockSpec((_NW * _M,), lambda: (0,)),
            pl.BlockSpec((_NW * _L,), lambda: (0,)),
            pl.BlockSpec((_B, 1, _M), lambda: (0, 0, 0)),
            pl.BlockSpec((1, 1), lambda: (0, 0), memory_space=pltpu.SMEM),
        ],
        out_specs=pl.BlockSpec(memory_space=pltpu.SMEM),
        out_shape=jax.ShapeDtypeStruct((1, 1), jnp.float32),
    )(colmin_sc, rowsum_sc, colmin_tc, sx_tc)
    return out[0, 0]


def kernel(x, y):
    xt_flat = jnp.transpose(x, (0, 2, 1)).reshape(-1)
    yt_flat = jnp.transpose(y, (0, 2, 1)).reshape(-1).astype(jnp.bfloat16)
    yt = jnp.transpose(y, (0, 2, 1))
    colmin_sc, rowsum_sc = _sc_chamfer(xt_flat, yt_flat)
    colmin_tc, sx_tc = _tc_chamfer(x, yt)
    return _combine(colmin_sc, rowsum_sc, colmin_tc, sx_tc)
